# baseline (speedup 1.0000x reference)
import functools

import jax
import jax.numpy as jnp
from jax import lax
from jax.experimental import pallas as pl
from jax.experimental.pallas import tpu as pltpu
from jax.experimental.pallas import tpu_sc as plsc

N_NODES = 10000
N_EDGES = 320000
D_IN = 128
D_OUT = 128

NC = 2
NS = 16
L = 16

D0 = 56
DH = D0 + L
C = 64
N_CHUNKS = 5120
CH_PER_W = N_CHUNKS // NS
NBUF = 5
IDX_BLK = 40
N_BLK_W = CH_PER_W // IDX_BLK
HBM_SLOTS = 2
E_PAD = N_CHUNKS * C
N_PAD_ROWS = 240
N_T = N_NODES + N_PAD_ROWS
ZB_ROWS = C
N_ZCH = N_T // ZB_ROWS

ROWS_PER_SUB = N_T // NS
STAGE_MAIN = 640
STAGE_LAST = N_NODES - (NS - 1) * STAGE_MAIN


def _sc_accumulate(xh, src2d, dst2d):
    mesh = plsc.VectorSubcoreMesh(core_axis_name="c", subcore_axis_name="s")

    @functools.partial(
        pl.kernel,
        mesh=mesh,
        compiler_params=pltpu.CompilerParams(use_tc_tiling_on_sc=False),
        out_type=jax.ShapeDtypeStruct((NC, N_T, DH), jnp.float32),
        scratch_types=[
            pltpu.VMEM((IDX_BLK, C), jnp.int32),
            pltpu.VMEM((IDX_BLK, C), jnp.int32),
            [pltpu.VMEM((C, DH), jnp.float32)] * NBUF,
            pltpu.VMEM_SHARED((N_NODES, DH), jnp.float32),
            pltpu.VMEM_SHARED((N_T, DH), jnp.float32),
            [pltpu.SemaphoreType.DMA] * NBUF,
            pltpu.SemaphoreType.DMA,
        ],
    )
    def sc_kernel(xh_hbm, src_hbm, dst_hbm, out_hbm, src_v, dst_v,
                  rows, table, acc, gsem, ssem):
        c = lax.axis_index("c")
        s = lax.axis_index("s")

        @pl.when(s < NS - 1)
        def _():
            pltpu.sync_copy(
                xh_hbm.at[c, pl.ds(s * STAGE_MAIN, STAGE_MAIN)],
                table.at[pl.ds(s * STAGE_MAIN, STAGE_MAIN)],
            )

        @pl.when(s == NS - 1)
        def _():
            pltpu.sync_copy(
                xh_hbm.at[c, pl.ds((NS - 1) * STAGE_MAIN, STAGE_LAST)],
                table.at[pl.ds((NS - 1) * STAGE_MAIN, STAGE_LAST)],
            )

        zvec = jnp.zeros((L,), jnp.float32)

        def zero_rows(k, _):
            i = k // (DH // L)
            j = k % (DH // L)
            rows[0][i, pl.ds(j * L, L)] = zvec
            return 0

        lax.fori_loop(0, ZB_ROWS * (DH // L), zero_rows, 0)

        def zero_acc(t, _):
            ch = s + NS * t
            pltpu.sync_copy(rows[0], acc.at[pl.ds(ch * ZB_ROWS, ZB_ROWS)])
            return 0

        lax.fori_loop(0, N_ZCH // NS, zero_acc, 0)
        plsc.subcore_barrier()

        xh_c = xh_hbm.at[c]
        src_for_slot = [xh_c if b < HBM_SLOTS else table
                        for b in range(NBUF)]

        def start_gather(ch, b):
            return pltpu.async_copy(src_for_slot[b].at[src_v.at[ch]],
                                    rows[b], gsem[b])

        def wait_gather(ch, b):
            pltpu.make_async_copy(src_for_slot[b].at[src_v.at[ch]],
                                  rows[b], gsem[b]).wait()

        def idx_block(h, _):
            base = s * CH_PER_W + h * IDX_BLK
            pltpu.sync_copy(src_hbm.at[pl.ds(base, IDX_BLK)], src_v)
            pltpu.sync_copy(dst_hbm.at[pl.ds(base, IDX_BLK)], dst_v)
            for b in range(NBUF - 1):
                start_gather(b, b)

            def chunk_group(k, _):
                for b in range(NBUF):
                    ch = NBUF * k + b
                    wait_gather(ch, b)
                    sc = pltpu.async_copy(rows[b], acc.at[dst_v.at[ch]],
                                          ssem, add=True)
                    sc.wait()
                    @pl.when(ch + NBUF - 1 < IDX_BLK)
                    def _():
                        start_gather(ch + NBUF - 1, (b + NBUF - 1) % NBUF)
                return 0

            lax.fori_loop(0, IDX_BLK // NBUF, chunk_group, 0)
            return 0

        lax.fori_loop(0, N_BLK_W, idx_block, 0)
        plsc.subcore_barrier()

        pltpu.sync_copy(
            acc.at[pl.ds(s * ROWS_PER_SUB, ROWS_PER_SUB)],
            out_hbm.at[c, pl.ds(s * ROWS_PER_SUB, ROWS_PER_SUB)],
        )

    return sc_kernel(xh, src2d, dst2d)


R_BLK = 400
N_BLKS = N_NODES // R_BLK


def _tc_body(x_ref, p_ref, w_ref, b_ref, o_ref):
    x = x_ref[...]
    deg = jnp.maximum(p_ref[0][:, D0:D0 + 1], 1.0)
    neigh_a = p_ref[0][:, :D0] / deg
    neigh_b = p_ref[1][...] / deg
    wx = w_ref[:, :D_IN]
    wn_a = w_ref[:, D_IN:D_IN + D0]
    wn_b = w_ref[:, D_IN + D0:]
    acc = lax.dot_general(x, wx, (((1,), (1,)), ((), ())),
                          preferred_element_type=jnp.float32)
    acc = acc + lax.dot_general(neigh_a, wn_a, (((1,), (1,)), ((), ())),
                                preferred_element_type=jnp.float32)
    acc = acc + lax.dot_general(neigh_b, wn_b, (((1,), (1,)), ((), ())),
                                preferred_element_type=jnp.float32)
    o_ref[...] = jnp.maximum(acc + b_ref[...], 0.0)


def _tc_linear(x, partials, W, b2d):
    return pl.pallas_call(
        _tc_body,
        grid=(N_BLKS,),
        in_specs=[
            pl.BlockSpec((R_BLK, D_IN), lambda i: (i, 0)),
            pl.BlockSpec((NC, R_BLK, DH), lambda i: (0, i, 0)),
            pl.BlockSpec((D_OUT, 2 * D_IN), lambda i: (0, 0)),
            pl.BlockSpec((1, D_OUT), lambda i: (0, 0)),
        ],
        out_specs=pl.BlockSpec((R_BLK, D_OUT), lambda i: (i, 0)),
        out_shape=jax.ShapeDtypeStruct((N_NODES, D_OUT), jnp.float32),
    )(x, partials, W, b2d)


@jax.jit
def kernel(x, edge_index, W, b):
    src = edge_index[0].astype(jnp.int32)
    dst = edge_index[1].astype(jnp.int32)

    n_pad = E_PAD - N_EDGES
    pad_src = jnp.zeros((n_pad,), jnp.int32)
    pad_dst = N_NODES + (jnp.arange(n_pad, dtype=jnp.int32) % N_PAD_ROWS)
    src2d = jnp.concatenate([src, pad_src]).reshape(N_CHUNKS, C)
    dst2d = jnp.concatenate([dst, pad_dst]).reshape(N_CHUNKS, C)

    ones = jnp.ones((N_NODES, L), jnp.float32)
    xh = jnp.stack(
        [jnp.concatenate([x[:, :D0], ones], axis=1), x[:, D0:]])

    partials = _sc_accumulate(xh, src2d, dst2d)
    return _tc_linear(x, partials, W, b.reshape(1, D_OUT))


# speedup vs baseline: 1.6700x; 1.6700x over previous
import functools

import jax
import jax.numpy as jnp
from jax import lax
from jax.experimental import pallas as pl
from jax.experimental.pallas import tpu as pltpu
from jax.experimental.pallas import tpu_sc as plsc

N_NODES = 10000
N_EDGES = 320000
D_IN = 128
D_OUT = 128

NC = 2
NS = 16
L = 16

D0 = 56
DH = D0 + L
C = 64
N_CHUNKS = 5120
CH_PER_W = N_CHUNKS // NS
NBUF = 5
IDX_BLK = 40
N_BLK_W = CH_PER_W // IDX_BLK
HBM_SLOTS = 0
E_PAD = N_CHUNKS * C
N_PAD_ROWS = 240
N_T = N_NODES + N_PAD_ROWS
ZB_ROWS = C
N_ZCH = N_T // ZB_ROWS

ROWS_PER_SUB = N_T // NS
STAGE_MAIN = 640
STAGE_LAST = N_NODES - (NS - 1) * STAGE_MAIN


def _sc_accumulate(xh, src2d, dst2d):
    mesh = plsc.VectorSubcoreMesh(core_axis_name="c", subcore_axis_name="s")

    @functools.partial(
        pl.kernel,
        mesh=mesh,
        compiler_params=pltpu.CompilerParams(use_tc_tiling_on_sc=False),
        out_type=jax.ShapeDtypeStruct((NC, N_T, DH), jnp.float32),
        scratch_types=[
            pltpu.VMEM((IDX_BLK, C), jnp.int32),
            pltpu.VMEM((IDX_BLK, C), jnp.int32),
            [pltpu.VMEM((C, DH), jnp.float32)] * NBUF,
            pltpu.VMEM_SHARED((N_NODES, DH), jnp.float32),
            pltpu.VMEM_SHARED((N_T, DH), jnp.float32),
            [pltpu.SemaphoreType.DMA] * NBUF,
            pltpu.SemaphoreType.DMA,
        ],
    )
    def sc_kernel(xh_hbm, src_hbm, dst_hbm, out_hbm, src_v, dst_v,
                  rows, table, acc, gsem, ssem):
        c = lax.axis_index("c")
        s = lax.axis_index("s")

        @pl.when(s < NS - 1)
        def _():
            pltpu.sync_copy(
                xh_hbm.at[c, pl.ds(s * STAGE_MAIN, STAGE_MAIN)],
                table.at[pl.ds(s * STAGE_MAIN, STAGE_MAIN)],
            )

        @pl.when(s == NS - 1)
        def _():
            pltpu.sync_copy(
                xh_hbm.at[c, pl.ds((NS - 1) * STAGE_MAIN, STAGE_LAST)],
                table.at[pl.ds((NS - 1) * STAGE_MAIN, STAGE_LAST)],
            )

        zvec = jnp.zeros((L,), jnp.float32)

        def zero_rows(k, _):
            i = k // (DH // L)
            j = k % (DH // L)
            rows[0][i, pl.ds(j * L, L)] = zvec
            return 0

        lax.fori_loop(0, ZB_ROWS * (DH // L), zero_rows, 0)

        def zero_acc(t, _):
            ch = s + NS * t
            pltpu.sync_copy(rows[0], acc.at[pl.ds(ch * ZB_ROWS, ZB_ROWS)])
            return 0

        lax.fori_loop(0, N_ZCH // NS, zero_acc, 0)
        plsc.subcore_barrier()

        xh_c = xh_hbm.at[c]
        src_for_slot = [xh_c if b < HBM_SLOTS else table
                        for b in range(NBUF)]

        def start_gather(ch, b):
            return pltpu.async_copy(src_for_slot[b].at[src_v.at[ch]],
                                    rows[b], gsem[b])

        def wait_gather(ch, b):
            pltpu.make_async_copy(src_for_slot[b].at[src_v.at[ch]],
                                  rows[b], gsem[b]).wait()

        def idx_block(h, _):
            base = s * CH_PER_W + h * IDX_BLK
            pltpu.sync_copy(src_hbm.at[pl.ds(base, IDX_BLK)], src_v)
            pltpu.sync_copy(dst_hbm.at[pl.ds(base, IDX_BLK)], dst_v)
            for b in range(NBUF - 1):
                start_gather(b, b)

            def chunk_group(k, _):
                for b in range(NBUF):
                    ch = NBUF * k + b
                    wait_gather(ch, b)
                    sc = pltpu.async_copy(rows[b], acc.at[dst_v.at[ch]],
                                          ssem, add=True)
                    sc.wait()
                    @pl.when(ch + NBUF - 1 < IDX_BLK)
                    def _():
                        start_gather(ch + NBUF - 1, (b + NBUF - 1) % NBUF)
                return 0

            lax.fori_loop(0, IDX_BLK // NBUF, chunk_group, 0)
            return 0

        lax.fori_loop(0, N_BLK_W, idx_block, 0)
        plsc.subcore_barrier()

        pltpu.sync_copy(
            acc.at[pl.ds(s * ROWS_PER_SUB, ROWS_PER_SUB)],
            out_hbm.at[c, pl.ds(s * ROWS_PER_SUB, ROWS_PER_SUB)],
        )

    return sc_kernel(xh, src2d, dst2d)


R_BLK = 400
N_BLKS = N_NODES // R_BLK


def _tc_self_body(x_ref, w_ref, b_ref, o_ref):
    wx = w_ref[:, :D_IN]
    acc = lax.dot_general(x_ref[...], wx, (((1,), (1,)), ((), ())),
                          preferred_element_type=jnp.float32)
    o_ref[...] = acc + b_ref[...]


def _tc_self(x, W, b2d):
    return pl.pallas_call(
        _tc_self_body,
        grid=(N_BLKS,),
        in_specs=[
            pl.BlockSpec((R_BLK, D_IN), lambda i: (i, 0)),
            pl.BlockSpec((D_OUT, 2 * D_IN), lambda i: (0, 0)),
            pl.BlockSpec((1, D_OUT), lambda i: (0, 0)),
        ],
        out_specs=pl.BlockSpec((R_BLK, D_OUT), lambda i: (i, 0)),
        out_shape=jax.ShapeDtypeStruct((N_NODES, D_OUT), jnp.float32),
    )(x, W, b2d)


def _tc_body(y_ref, p_ref, w_ref, o_ref):
    deg = jnp.maximum(p_ref[0][:, D0:D0 + 1], 1.0)
    neigh_a = p_ref[0][:, :D0] / deg
    neigh_b = p_ref[1][...] / deg
    wn_a = w_ref[:, D_IN:D_IN + D0]
    wn_b = w_ref[:, D_IN + D0:]
    acc = y_ref[...]
    acc = acc + lax.dot_general(neigh_a, wn_a, (((1,), (1,)), ((), ())),
                                preferred_element_type=jnp.float32)
    acc = acc + lax.dot_general(neigh_b, wn_b, (((1,), (1,)), ((), ())),
                                preferred_element_type=jnp.float32)
    o_ref[...] = jnp.maximum(acc, 0.0)


def _tc_linear(y, partials, W):
    return pl.pallas_call(
        _tc_body,
        grid=(N_BLKS,),
        in_specs=[
            pl.BlockSpec((R_BLK, D_OUT), lambda i: (i, 0)),
            pl.BlockSpec((NC, R_BLK, DH), lambda i: (0, i, 0)),
            pl.BlockSpec((D_OUT, 2 * D_IN), lambda i: (0, 0)),
        ],
        out_specs=pl.BlockSpec((R_BLK, D_OUT), lambda i: (i, 0)),
        out_shape=jax.ShapeDtypeStruct((N_NODES, D_OUT), jnp.float32),
    )(y, partials, W)


@jax.jit
def kernel(x, edge_index, W, b):
    src = edge_index[0].astype(jnp.int32)
    dst = edge_index[1].astype(jnp.int32)

    n_pad = E_PAD - N_EDGES
    pad_src = jnp.zeros((n_pad,), jnp.int32)
    pad_dst = N_NODES + (jnp.arange(n_pad, dtype=jnp.int32) % N_PAD_ROWS)
    src2d = jnp.concatenate([src, pad_src]).reshape(N_CHUNKS, C)
    dst2d = jnp.concatenate([dst, pad_dst]).reshape(N_CHUNKS, C)

    ones = jnp.ones((N_NODES, L), jnp.float32)
    xh = jnp.stack(
        [jnp.concatenate([x[:, :D0], ones], axis=1), x[:, D0:]])

    y = _tc_self(x, W, b.reshape(1, D_OUT))
    partials = _sc_accumulate(xh, src2d, dst2d)
    return _tc_linear(y, partials, W)
